# baseline (speedup 1.0000x reference)
import functools

import jax
import jax.numpy as jnp
from jax import lax
from jax.experimental import pallas as pl
from jax.experimental.pallas import tpu as pltpu
from jax.experimental.pallas import tpu_sc as plsc

DIM = 64
B = 16384
NREQ = 2 * B
NC = 2
NS = 16
NW = NC * NS
BPW = B // NW
NSTR_TOT = 7813
SPW = 245
NROW = 32
RING = 5

_mesh = plsc.VectorSubcoreMesh(core_axis_name="c", subcore_axis_name="s")


@functools.partial(
    pl.kernel,
    out_type=jax.ShapeDtypeStruct((NREQ * DIM,), jnp.float32),
    mesh=_mesh,
    compiler_params=pltpu.CompilerParams(
        needs_layout_passes=False, use_tc_tiling_on_sc=True),
    scratch_types=[
        pltpu.VMEM((B,), jnp.int32),
        pltpu.VMEM((NREQ,), jnp.int32),
        pltpu.VMEM((DIM, 512), jnp.float32),
        pltpu.VMEM((DIM, 512), jnp.float32),
        pltpu.VMEM((NROW, DIM), jnp.float32),
        pltpu.SMEM((256,), jnp.int32),
        pltpu.SMEM((257,), jnp.int32),
        pltpu.SMEM((256,), jnp.int32),
        pltpu.SMEM((1,), jnp.int32),
        pltpu.SemaphoreType.DMA,
        pltpu.SemaphoreType.DMA,
        pltpu.SemaphoreType.DMA,
        pltpu.SemaphoreType.DMA,
    ],
)
def _extract_sc(src_h, dst_h, entT_h, inter_h,
                idx_v, hit_v, sbufA, sbufB, rowbuf,
                cnt_s, off_s, pos_s, jctr, sem_i, semA, semB, sem_o):
    cid = lax.axis_index("c")
    sid = lax.axis_index("s")
    wid = sid * NC + cid
    w_lo = wid * SPW
    nstr = jnp.minimum(SPW, NSTR_TOT - w_lo)

    lane = lax.broadcasted_iota(jnp.int32, (16,), 0)
    lane0 = lane == 0

    def fire(p, buf, sm):
        sbase = jnp.minimum(4 * p, nstr - 4)
        pltpu.make_async_copy(
            entT_h.at[:, pl.ds((w_lo + sbase) * 128, 512)], buf, sm).start()

    def wait_slab(buf, sm):
        pltpu.make_async_copy(
            entT_h.at[:, pl.ds(0, 512)], buf, sm).wait()

    fire(0, sbufA, semA)

    def zero_body(l, carry):
        cnt_s[l] = 0
        return carry

    lax.fori_loop(0, 256, zero_body, 0)

    def count_body(k, carry):
        e = idx_v[pl.ds(pl.multiple_of(k * 16, 16), 16)]
        local = (e >> 7) - w_lo
        mi = ((local >= 0) & (local < nstr)).astype(jnp.int32)
        c = plsc.cumsum(mi)

        @pl.when(c[15] > 0)
        def _():
            for i in range(16):
                @pl.when(mi[i] != 0)
                def _():
                    l = local[i]
                    cnt_s[l] = cnt_s[l] + 1
        return carry

    pltpu.async_copy(dst_h, idx_v, sem_i).wait()
    lax.fori_loop(0, B // 16, count_body, 0)
    pltpu.async_copy(src_h, idx_v, sem_i).wait()
    lax.fori_loop(0, B // 16, count_body, 0)

    def off_body(l, run):
        off_s[l] = run
        pos_s[l] = run
        return run + cnt_s[l]

    total = lax.fori_loop(0, 256, off_body, 0)
    off_s[256] = total

    def make_place(slot_base):
        def place_body(k, carry):
            e = idx_v[pl.ds(pl.multiple_of(k * 16, 16), 16)]
            local = (e >> 7) - w_lo
            mi = ((local >= 0) & (local < nstr)).astype(jnp.int32)
            c = plsc.cumsum(mi)
            val = (local << 22) | ((e & 127) << 15) | (slot_base + k * 16 + lane)

            @pl.when(c[15] > 0)
            def _():
                for i in range(16):
                    @pl.when(mi[i] != 0)
                    def _():
                        l = local[i]
                        p = pos_s[l]
                        plsc.store_scatter(
                            hit_v, [jnp.full((16,), p, jnp.int32)],
                            jnp.full((16,), val[i], jnp.int32), mask=lane0)
                        pos_s[l] = p + 1
            return carry
        return place_body

    pltpu.async_copy(dst_h, idx_v, sem_i).wait()
    lax.fori_loop(0, B // 16, make_place(0), 0)
    pltpu.async_copy(src_h, idx_v, sem_i).wait()
    lax.fori_loop(0, B // 16, make_place(B), 0)
    jctr[0] = 0

    def process(p, buf):
        sbase = jnp.minimum(4 * p, nstr - 4)
        lo = off_s[4 * p]
        hi = off_s[4 * p + 4]

        def chunk_body(k, carry):
            pos16 = k * 16 + lane
            vals = hit_v[pl.ds(pl.multiple_of(k * 16, 16), 16)]
            slotv = vals & (NREQ - 1)
            eov = ((vals >> 15) & 127) + ((vals >> 22) - sbase) * 128
            validm = ((pos16 >= lo) & (pos16 < hi)).astype(jnp.int32)
            for i in range(16):
                @pl.when(validm[i] != 0)
                def _():
                    colv = jnp.full((16,), eov[i], jnp.int32)
                    j = jctr[0]
                    jj = j & (NROW - 1)

                    @pl.when(j >= NROW)
                    def _():
                        pltpu.make_async_copy(
                            rowbuf.at[0], inter_h.at[pl.ds(0, DIM)],
                            sem_o).wait()

                    for q in range(DIM // 16):
                        dv = plsc.load_gather(buf, [q * 16 + lane, colv])
                        rowbuf[jj, pl.ds(q * 16, 16)] = dv
                    pltpu.make_async_copy(
                        rowbuf.at[jj],
                        inter_h.at[pl.ds(slotv[i] * DIM, DIM)],
                        sem_o).start()
                    jctr[0] = j + 1
            return carry

        lax.fori_loop(lo >> 4, (hi + 15) >> 4, chunk_body, 0)

    npair = (nstr + 3) >> 2

    def pair_body(pp, carry):
        p0 = 2 * pp
        p1 = p0 + 1

        @pl.when(p1 < npair)
        def _():
            fire(p1, sbufB, semB)

        wait_slab(sbufA, semA)
        process(p0, sbufA)

        @pl.when(p1 < npair)
        def _():
            @pl.when(p1 + 1 < npair)
            def _():
                fire(p1 + 1, sbufA, semA)

            wait_slab(sbufB, semB)
            process(p1, sbufB)
        return carry

    lax.fori_loop(0, (npair + 1) // 2, pair_body, 0)

    def drain_body(i, carry):
        pltpu.make_async_copy(
            rowbuf.at[0], inter_h.at[pl.ds(0, DIM)], sem_o).wait()
        return carry

    lax.fori_loop(0, jnp.minimum(jctr[0], NROW), drain_body, 0)


@functools.partial(
    pl.kernel,
    out_type=jax.ShapeDtypeStruct((B,), jnp.float32),
    mesh=_mesh,
    compiler_params=pltpu.CompilerParams(
        needs_layout_passes=False, use_tc_tiling_on_sc=False),
    scratch_types=[
        pltpu.VMEM((4, 128), jnp.int32),
        pltpu.VMEM((BPW * DIM,), jnp.float32),
        pltpu.VMEM((BPW * DIM,), jnp.float32),
        pltpu.VMEM((BPW, DIM), jnp.float32),
        pltpu.VMEM((BPW,), jnp.float32),
        pltpu.SemaphoreType.DMA,
    ],
)
def _score_sc(rel_h, relemb_h, inter_h, out_h,
              idx_r, dbuf, sbuf, rbuf, out_v, sem):
    cid = lax.axis_index("c")
    sid = lax.axis_index("s")
    wid = sid * NC + cid
    base = wid * BPW

    cps = []
    for ch in range(4):
        cps.append(pltpu.async_copy(
            rel_h.at[pl.ds(base + ch * 128, 128)], idx_r.at[ch], sem))
    cps.append(pltpu.async_copy(
        inter_h.at[pl.ds(base * DIM, BPW * DIM)], dbuf, sem))
    cps.append(pltpu.async_copy(
        inter_h.at[pl.ds((B + base) * DIM, BPW * DIM)], sbuf, sem))
    for cp in cps:
        cp.wait()

    gs = []
    for ch in range(4):
        gs.append(pltpu.async_copy(
            relemb_h.at[idx_r.at[ch]], rbuf.at[pl.ds(ch * 128, 128)], sem))
    for cp in gs:
        cp.wait()

    lane = lax.broadcasted_iota(jnp.int32, (16,), 0)
    lane_eq = [lane == i for i in range(16)]

    def group_body(g, carry):
        res = jnp.zeros((16,), jnp.float32)
        rbase = g * 16
        for i in range(16):
            r = rbase + i
            acc = jnp.zeros((16,), jnp.float32)
            for q in range(DIM // 16):
                t = (dbuf[pl.ds(r * DIM + q * 16, 16)]
                     - sbuf[pl.ds(r * DIM + q * 16, 16)]
                     - rbuf[r, pl.ds(q * 16, 16)] + 1e-30)
                acc = acc + t * t
            res = jnp.where(lane_eq[i], jnp.sum(acc), res)
        acc = jnp.maximum(res, 1e-30)
        i32 = plsc.bitcast(acc, jnp.int32)
        i32 = 0x5F3759DF - (i32 >> 1)
        y = plsc.bitcast(i32, jnp.float32)
        half = acc * 0.5
        for _ in range(3):
            y = y * (1.5 - half * y * y)
        out_v[pl.ds(pl.multiple_of(rbase, 16), 16)] = acc * y
        return carry

    lax.fori_loop(0, BPW // 16, group_body, 0)
    pltpu.sync_copy(out_v, out_h.at[pl.ds(base, BPW)])


def kernel(src, rel, dst, ent_embed, rel_embed):
    inter = _extract_sc(src, dst, ent_embed.T)
    return _score_sc(rel, rel_embed, inter)


# speedup vs baseline: 2.4185x; 2.4185x over previous
import functools

import jax
import jax.numpy as jnp
from jax import lax
from jax.experimental import pallas as pl
from jax.experimental.pallas import tpu as pltpu
from jax.experimental.pallas import tpu_sc as plsc

DIM = 64
B = 16384
NREQ = 2 * B
NC = 2
NS = 16
NW = NC * NS
BPW = B // NW
NSTR_TOT = 7813
SPW = 245
NROW = 32
RING = 5

_mesh = plsc.VectorSubcoreMesh(core_axis_name="c", subcore_axis_name="s")


@functools.partial(
    pl.kernel,
    out_type=jax.ShapeDtypeStruct((NREQ * DIM,), jnp.float32),
    mesh=_mesh,
    compiler_params=pltpu.CompilerParams(
        needs_layout_passes=False, use_tc_tiling_on_sc=True),
    scratch_types=[
        pltpu.VMEM((NREQ,), jnp.int32),
        pltpu.VMEM((NREQ,), jnp.int32),
        pltpu.VMEM((DIM, 512), jnp.float32),
        pltpu.VMEM((DIM, 256), jnp.float32),
        pltpu.VMEM((NROW, DIM), jnp.float32),
        pltpu.SMEM((256,), jnp.int32),
        pltpu.SMEM((257,), jnp.int32),
        pltpu.SMEM((256,), jnp.int32),
        pltpu.SMEM((1,), jnp.int32),
        pltpu.SemaphoreType.DMA,
        pltpu.SemaphoreType.DMA,
        pltpu.SemaphoreType.DMA,
        pltpu.SemaphoreType.DMA,
    ],
)
def _extract_sc(src_h, dst_h, entT_h, inter_h,
                req_v, hit_v, sbufA, sbufB, rowbuf,
                cnt_s, off_s, pos_s, jctr, sem_i, semA, semB, sem_o):
    cid = lax.axis_index("c")
    sid = lax.axis_index("s")
    wid = sid * NC + cid
    w_lo = wid * SPW
    nstr = jnp.minimum(SPW, NSTR_TOT - w_lo)

    lane = lax.broadcasted_iota(jnp.int32, (16,), 0)
    lane0 = lane == 0

    def fireA(g, sm):
        sbase = jnp.minimum(6 * g, nstr - 4)
        pltpu.make_async_copy(
            entT_h.at[:, pl.ds((w_lo + sbase) * 128, 512)], sbufA, sm).start()

    def fireB(g, sm):
        sbase = jnp.minimum(6 * g + 4, nstr - 2)
        pltpu.make_async_copy(
            entT_h.at[:, pl.ds((w_lo + sbase) * 128, 256)], sbufB, sm).start()

    def waitA(sm):
        pltpu.make_async_copy(
            entT_h.at[:, pl.ds(0, 512)], sbufA, sm).wait()

    def waitB(sm):
        pltpu.make_async_copy(
            entT_h.at[:, pl.ds(0, 256)], sbufB, sm).wait()

    def compact_half(slot_base, fill0):
        def compact_body(k, fill):
            e = hit_v[pl.ds(pl.multiple_of(slot_base + k * 16, 16), 16)]
            local = (e >> 7) - w_lo
            m = (local >= 0) & (local < nstr)
            c = plsc.cumsum(m.astype(jnp.int32))
            wr = fill + c - 1
            val = (local << 22) | ((e & 127) << 15) | (slot_base + k * 16 + lane)
            plsc.store_scatter(req_v, [wr], val, mask=m)
            return fill + c[15]

        return lax.fori_loop(0, B // 16, compact_body, fill0)

    ci1 = pltpu.async_copy(dst_h, hit_v.at[pl.ds(0, B)], sem_i)
    ci2 = pltpu.async_copy(src_h, hit_v.at[pl.ds(B, B)], sem_i)
    ci1.wait()
    ci2.wait()
    nreq = compact_half(0, 0)
    nreq = compact_half(B, nreq)
    nreqv = (nreq + 15) >> 4

    def zero_body(l, carry):
        cnt_s[l] = 0
        return carry

    lax.fori_loop(0, 256, zero_body, 0)

    def count_body(kv, carry):
        reqv = req_v[pl.ds(pl.multiple_of(kv * 16, 16), 16)]
        lv = reqv >> 22
        valid = ((kv * 16 + lane) < nreq).astype(jnp.int32)
        for i in range(16):
            @pl.when(valid[i] != 0)
            def _():
                l = lv[i]
                cnt_s[l] = cnt_s[l] + 1
        return carry

    lax.fori_loop(0, nreqv, count_body, 0)

    def off_body(l, run):
        off_s[l] = run
        pos_s[l] = run
        return run + cnt_s[l]

    total = lax.fori_loop(0, 256, off_body, 0)
    off_s[256] = total

    def place_body(kv, carry):
        reqv = req_v[pl.ds(pl.multiple_of(kv * 16, 16), 16)]
        lv = reqv >> 22
        valid = ((kv * 16 + lane) < nreq).astype(jnp.int32)
        for i in range(16):
            @pl.when(valid[i] != 0)
            def _():
                l = lv[i]
                p = pos_s[l]
                plsc.store_scatter(
                    hit_v, [jnp.full((16,), p, jnp.int32)],
                    jnp.full((16,), reqv[i], jnp.int32), mask=lane0)
                pos_s[l] = p + 1
        return carry

    lax.fori_loop(0, nreqv, place_body, 0)
    jctr[0] = 0

    def process(lo, hi, sbase, buf):

        def chunk_body(k, carry):
            pos16 = k * 16 + lane
            vals = hit_v[pl.ds(pl.multiple_of(k * 16, 16), 16)]
            slotv = vals & (NREQ - 1)
            eov = ((vals >> 15) & 127) + ((vals >> 22) - sbase) * 128
            validm = ((pos16 >= lo) & (pos16 < hi)).astype(jnp.int32)
            for i in range(16):
                @pl.when(validm[i] != 0)
                def _():
                    colv = jnp.full((16,), eov[i], jnp.int32)
                    j = jctr[0]
                    jj = j & (NROW - 1)

                    @pl.when(j >= NROW)
                    def _():
                        pltpu.make_async_copy(
                            rowbuf.at[0], inter_h.at[pl.ds(0, DIM)],
                            sem_o).wait()

                    for q in range(DIM // 16):
                        dv = plsc.load_gather(buf, [q * 16 + lane, colv])
                        rowbuf[jj, pl.ds(q * 16, 16)] = dv
                    pltpu.make_async_copy(
                        rowbuf.at[jj],
                        inter_h.at[pl.ds(slotv[i] * DIM, DIM)],
                        sem_o).start()
                    jctr[0] = j + 1
            return carry

        lax.fori_loop(lo >> 4, (hi + 15) >> 4, chunk_body, 0)

    ngrp = (nstr + 5) // 6
    fireA(0, semA)
    fireB(0, semB)

    def grp_body(g, carry):
        waitA(semA)
        process(off_s[jnp.minimum(6 * g, 256)],
                off_s[jnp.minimum(6 * g + 4, 256)],
                jnp.minimum(6 * g, nstr - 4), sbufA)

        @pl.when(g + 1 < ngrp)
        def _():
            fireA(g + 1, semA)

        waitB(semB)
        process(off_s[jnp.minimum(6 * g + 4, 256)],
                off_s[jnp.minimum(6 * g + 6, 256)],
                jnp.minimum(6 * g + 4, nstr - 2), sbufB)

        @pl.when(g + 1 < ngrp)
        def _():
            fireB(g + 1, semB)
        return carry

    lax.fori_loop(0, ngrp, grp_body, 0)

    def drain_body(i, carry):
        pltpu.make_async_copy(
            rowbuf.at[0], inter_h.at[pl.ds(0, DIM)], sem_o).wait()
        return carry

    lax.fori_loop(0, jnp.minimum(jctr[0], NROW), drain_body, 0)


@functools.partial(
    pl.kernel,
    out_type=jax.ShapeDtypeStruct((B,), jnp.float32),
    mesh=_mesh,
    compiler_params=pltpu.CompilerParams(
        needs_layout_passes=False, use_tc_tiling_on_sc=False),
    scratch_types=[
        pltpu.VMEM((4, 128), jnp.int32),
        pltpu.VMEM((BPW * DIM,), jnp.float32),
        pltpu.VMEM((BPW * DIM,), jnp.float32),
        pltpu.VMEM((BPW, DIM), jnp.float32),
        pltpu.VMEM((BPW,), jnp.float32),
        pltpu.SemaphoreType.DMA,
    ],
)
def _score_sc(rel_h, relemb_h, inter_h, out_h,
              idx_r, dbuf, sbuf, rbuf, out_v, sem):
    cid = lax.axis_index("c")
    sid = lax.axis_index("s")
    wid = sid * NC + cid
    base = wid * BPW

    cps = []
    for ch in range(4):
        cps.append(pltpu.async_copy(
            rel_h.at[pl.ds(base + ch * 128, 128)], idx_r.at[ch], sem))
    cps.append(pltpu.async_copy(
        inter_h.at[pl.ds(base * DIM, BPW * DIM)], dbuf, sem))
    cps.append(pltpu.async_copy(
        inter_h.at[pl.ds((B + base) * DIM, BPW * DIM)], sbuf, sem))
    for cp in cps:
        cp.wait()

    gs = []
    for ch in range(4):
        gs.append(pltpu.async_copy(
            relemb_h.at[idx_r.at[ch]], rbuf.at[pl.ds(ch * 128, 128)], sem))
    for cp in gs:
        cp.wait()

    lane = lax.broadcasted_iota(jnp.int32, (16,), 0)
    lane_eq = [lane == i for i in range(16)]

    def group_body(g, carry):
        res = jnp.zeros((16,), jnp.float32)
        rbase = g * 16
        for i in range(16):
            r = rbase + i
            acc = jnp.zeros((16,), jnp.float32)
            for q in range(DIM // 16):
                t = (dbuf[pl.ds(r * DIM + q * 16, 16)]
                     - sbuf[pl.ds(r * DIM + q * 16, 16)]
                     - rbuf[r, pl.ds(q * 16, 16)] + 1e-30)
                acc = acc + t * t
            res = jnp.where(lane_eq[i], jnp.sum(acc), res)
        acc = jnp.maximum(res, 1e-30)
        i32 = plsc.bitcast(acc, jnp.int32)
        i32 = 0x5F3759DF - (i32 >> 1)
        y = plsc.bitcast(i32, jnp.float32)
        half = acc * 0.5
        for _ in range(3):
            y = y * (1.5 - half * y * y)
        out_v[pl.ds(pl.multiple_of(rbase, 16), 16)] = acc * y
        return carry

    lax.fori_loop(0, BPW // 16, group_body, 0)
    pltpu.sync_copy(out_v, out_h.at[pl.ds(base, BPW)])


def kernel(src, rel, dst, ent_embed, rel_embed):
    inter = _extract_sc(src, dst, ent_embed.T)
    return _score_sc(rel, rel_embed, inter)


# speedup vs baseline: 2.6230x; 1.0846x over previous
import functools

import jax
import jax.numpy as jnp
from jax import lax
from jax.experimental import pallas as pl
from jax.experimental.pallas import tpu as pltpu
from jax.experimental.pallas import tpu_sc as plsc

DIM = 64
B = 16384
NREQ = 2 * B
NC = 2
NS = 16
NW = NC * NS
BPW = B // NW
NSTR_TOT = 7813
SPW = 245
NROW = 32
RING = 5

_mesh = plsc.VectorSubcoreMesh(core_axis_name="c", subcore_axis_name="s")


@functools.partial(
    pl.kernel,
    out_type=jax.ShapeDtypeStruct((NREQ * DIM,), jnp.float32),
    mesh=_mesh,
    compiler_params=pltpu.CompilerParams(
        needs_layout_passes=False, use_tc_tiling_on_sc=True),
    scratch_types=[
        pltpu.VMEM((B,), jnp.int32),
        pltpu.VMEM((NREQ,), jnp.int32),
        pltpu.VMEM((NREQ,), jnp.int32),
        pltpu.VMEM((DIM, 256), jnp.float32),
        pltpu.VMEM((DIM, 256), jnp.float32),
        pltpu.VMEM((NROW, DIM), jnp.float32),
        pltpu.SMEM((256,), jnp.int32),
        pltpu.SMEM((257,), jnp.int32),
        pltpu.SMEM((256,), jnp.int32),
        pltpu.SMEM((1,), jnp.int32),
        pltpu.SemaphoreType.DMA,
        pltpu.SemaphoreType.DMA,
        pltpu.SemaphoreType.DMA,
        pltpu.SemaphoreType.DMA,
    ],
)
def _extract_sc(src_h, dst_h, entT_h, inter_h,
                idx_v, req_v, hit_v, sbufA, sbufB, rowbuf,
                cnt_s, off_s, pos_s, jctr, sem_i, semA, semB, sem_o):
    cid = lax.axis_index("c")
    sid = lax.axis_index("s")
    wid = sid * NC + cid
    w_lo = wid * SPW
    nstr = jnp.minimum(SPW, NSTR_TOT - w_lo)

    lane = lax.broadcasted_iota(jnp.int32, (16,), 0)
    lane0 = lane == 0

    def fire(p, buf, sm):
        sbase = jnp.minimum(2 * p, nstr - 2)
        pltpu.make_async_copy(
            entT_h.at[:, pl.ds((w_lo + sbase) * 128, 256)], buf, sm).start()

    def wait_slab(buf, sm):
        pltpu.make_async_copy(
            entT_h.at[:, pl.ds(0, 256)], buf, sm).wait()

    fire(0, sbufA, semA)

    def compact_half(slot_base, fill0):
        def compact_body(k, fill):
            e = idx_v[pl.ds(pl.multiple_of(k * 16, 16), 16)]
            local = (e >> 7) - w_lo
            m = (local >= 0) & (local < nstr)
            c = plsc.cumsum(m.astype(jnp.int32))
            wr = fill + c - 1
            val = (local << 22) | ((e & 127) << 15) | (slot_base + k * 16 + lane)
            plsc.store_scatter(req_v, [wr], val, mask=m)
            return fill + c[15]

        return lax.fori_loop(0, B // 16, compact_body, fill0)

    pltpu.async_copy(dst_h, idx_v, sem_i).wait()
    nreq = compact_half(0, 0)
    pltpu.async_copy(src_h, idx_v, sem_i).wait()
    nreq = compact_half(B, nreq)
    nreqv = (nreq + 15) >> 4

    def zero_body(l, carry):
        cnt_s[l] = 0
        return carry

    lax.fori_loop(0, 256, zero_body, 0)

    def count_body(kv, carry):
        reqv = req_v[pl.ds(pl.multiple_of(kv * 16, 16), 16)]
        lv = reqv >> 22
        valid = ((kv * 16 + lane) < nreq).astype(jnp.int32)
        for i in range(16):
            @pl.when(valid[i] != 0)
            def _():
                l = lv[i]
                cnt_s[l] = cnt_s[l] + 1
        return carry

    lax.fori_loop(0, nreqv, count_body, 0)

    def off_body(l, run):
        off_s[l] = run
        pos_s[l] = run
        return run + cnt_s[l]

    total = lax.fori_loop(0, 256, off_body, 0)
    off_s[256] = total

    def place_body(kv, carry):
        reqv = req_v[pl.ds(pl.multiple_of(kv * 16, 16), 16)]
        lv = reqv >> 22
        valid = ((kv * 16 + lane) < nreq).astype(jnp.int32)
        for i in range(16):
            @pl.when(valid[i] != 0)
            def _():
                l = lv[i]
                p = pos_s[l]
                plsc.store_scatter(
                    hit_v, [jnp.full((16,), p, jnp.int32)],
                    jnp.full((16,), reqv[i], jnp.int32), mask=lane0)
                pos_s[l] = p + 1
        return carry

    lax.fori_loop(0, nreqv, place_body, 0)
    jctr[0] = 0

    def process(p, buf):
        sbase = jnp.minimum(2 * p, nstr - 2)
        lo = off_s[2 * p]
        hi = off_s[2 * p + 2]

        def chunk_body(k, carry):
            pos16 = k * 16 + lane
            vals = hit_v[pl.ds(pl.multiple_of(k * 16, 16), 16)]
            slotv = vals & (NREQ - 1)
            eov = ((vals >> 15) & 127) + ((vals >> 22) - sbase) * 128
            validm = ((pos16 >= lo) & (pos16 < hi)).astype(jnp.int32)
            for i in range(16):
                @pl.when(validm[i] != 0)
                def _():
                    colv = jnp.full((16,), eov[i], jnp.int32)
                    j = jctr[0]
                    jj = j & (NROW - 1)

                    @pl.when(j >= NROW)
                    def _():
                        pltpu.make_async_copy(
                            rowbuf.at[0], inter_h.at[pl.ds(0, DIM)],
                            sem_o).wait()

                    for q in range(DIM // 16):
                        dv = plsc.load_gather(buf, [q * 16 + lane, colv])
                        rowbuf[jj, pl.ds(q * 16, 16)] = dv
                    pltpu.make_async_copy(
                        rowbuf.at[jj],
                        inter_h.at[pl.ds(slotv[i] * DIM, DIM)],
                        sem_o).start()
                    jctr[0] = j + 1
            return carry

        lax.fori_loop(lo >> 4, (hi + 15) >> 4, chunk_body, 0)

    npair = (nstr + 1) >> 1

    def pair_body(pp, carry):
        p0 = 2 * pp
        p1 = p0 + 1

        @pl.when(p1 < npair)
        def _():
            fire(p1, sbufB, semB)

        wait_slab(sbufA, semA)
        process(p0, sbufA)

        @pl.when(p1 < npair)
        def _():
            @pl.when(p1 + 1 < npair)
            def _():
                fire(p1 + 1, sbufA, semA)

            wait_slab(sbufB, semB)
            process(p1, sbufB)
        return carry

    lax.fori_loop(0, (npair + 1) // 2, pair_body, 0)

    def drain_body(i, carry):
        pltpu.make_async_copy(
            rowbuf.at[0], inter_h.at[pl.ds(0, DIM)], sem_o).wait()
        return carry

    lax.fori_loop(0, jnp.minimum(jctr[0], NROW), drain_body, 0)


@functools.partial(
    pl.kernel,
    out_type=jax.ShapeDtypeStruct((B,), jnp.float32),
    mesh=_mesh,
    compiler_params=pltpu.CompilerParams(
        needs_layout_passes=False, use_tc_tiling_on_sc=False),
    scratch_types=[
        pltpu.VMEM((4, 128), jnp.int32),
        pltpu.VMEM((BPW * DIM,), jnp.float32),
        pltpu.VMEM((BPW * DIM,), jnp.float32),
        pltpu.VMEM((BPW, DIM), jnp.float32),
        pltpu.VMEM((BPW,), jnp.float32),
        pltpu.SemaphoreType.DMA,
    ],
)
def _score_sc(rel_h, relemb_h, inter_h, out_h,
              idx_r, dbuf, sbuf, rbuf, out_v, sem):
    cid = lax.axis_index("c")
    sid = lax.axis_index("s")
    wid = sid * NC + cid
    base = wid * BPW

    cps = []
    for ch in range(4):
        cps.append(pltpu.async_copy(
            rel_h.at[pl.ds(base + ch * 128, 128)], idx_r.at[ch], sem))
    cps.append(pltpu.async_copy(
        inter_h.at[pl.ds(base * DIM, BPW * DIM)], dbuf, sem))
    cps.append(pltpu.async_copy(
        inter_h.at[pl.ds((B + base) * DIM, BPW * DIM)], sbuf, sem))
    for cp in cps:
        cp.wait()

    gs = []
    for ch in range(4):
        gs.append(pltpu.async_copy(
            relemb_h.at[idx_r.at[ch]], rbuf.at[pl.ds(ch * 128, 128)], sem))
    for cp in gs:
        cp.wait()

    lane = lax.broadcasted_iota(jnp.int32, (16,), 0)
    lane_eq = [lane == i for i in range(16)]

    def group_body(g, carry):
        res = jnp.zeros((16,), jnp.float32)
        rbase = g * 16
        for i in range(16):
            r = rbase + i
            acc = jnp.zeros((16,), jnp.float32)
            for q in range(DIM // 16):
                t = (dbuf[pl.ds(r * DIM + q * 16, 16)]
                     - sbuf[pl.ds(r * DIM + q * 16, 16)]
                     - rbuf[r, pl.ds(q * 16, 16)] + 1e-30)
                acc = acc + t * t
            res = jnp.where(lane_eq[i], jnp.sum(acc), res)
        acc = jnp.maximum(res, 1e-30)
        i32 = plsc.bitcast(acc, jnp.int32)
        i32 = 0x5F3759DF - (i32 >> 1)
        y = plsc.bitcast(i32, jnp.float32)
        half = acc * 0.5
        for _ in range(3):
            y = y * (1.5 - half * y * y)
        out_v[pl.ds(pl.multiple_of(rbase, 16), 16)] = acc * y
        return carry

    lax.fori_loop(0, BPW // 16, group_body, 0)
    pltpu.sync_copy(out_v, out_h.at[pl.ds(base, BPW)])


def kernel(src, rel, dst, ent_embed, rel_embed):
    inter = _extract_sc(src, dst, ent_embed.T)
    return _score_sc(rel, rel_embed, inter)
